# Initial kernel scaffold; baseline (speedup 1.0000x reference)
import jax
import jax.numpy as jnp
from jax.experimental import pallas as pl


def kernel(cls, label_cls, pred_loc, label_loc, shape):
    raise NotImplementedError("write your pallas kernel here")


# speedup vs baseline: 5303.8954x; 5303.8954x over previous
import jax
import jax.numpy as jnp
from jax.experimental import pallas as pl
from jax.experimental.pallas import tpu as pltpu

GAMMA = 3.0
N = 625
NPAD = 640
B = 16


def _loss_kernel(cls1_ref, lab_ref, ploc_ref, lloc_ref, shp_ref,
                 f1_ref, f2_ref, cnt_ref):
    b = pl.program_id(0)

    @pl.when(b == 0)
    def _init():
        f1_ref[...] = jnp.zeros((1, 1), jnp.float32)
        f2_ref[...] = jnp.zeros((1, 1), jnp.float32)
        cnt_ref[...] = jnp.zeros((1, 1), jnp.float32)

    lab = lab_ref[0]
    m = lab > 0
    mf = m.astype(jnp.float32)

    cls1 = cls1_ref[0]
    pos_prob = jnp.exp(cls1)

    ploc = ploc_ref[0]
    lloc = lloc_ref[0]
    shp = shp_ref[...]

    sh0 = shp[0:1, :]
    sh1 = shp[1:2, :]
    sh2 = shp[2:3, :]
    sh3 = shp[3:4, :]

    def corners(loc):
        cx = loc[0:1, :] * sh2 + sh0
        cy = loc[1:2, :] * sh3 + sh1
        w = jnp.exp(loc[2:3, :]) * sh2
        h = jnp.exp(loc[3:4, :]) * sh3
        half_w = w * 0.5
        half_h = h * 0.5
        return cx - half_w, cy - half_h, cx + half_w, cy + half_h

    ax1, ay1, ax2, ay2 = corners(ploc)
    bx1, by1, bx2, by2 = corners(lloc)

    ix1 = jnp.maximum(ax1, bx1)
    iy1 = jnp.maximum(ay1, by1)
    ix2 = jnp.minimum(ax2, bx2)
    iy2 = jnp.minimum(ay2, by2)
    inter = jnp.maximum(ix2 - ix1, 0.0) * jnp.maximum(iy2 - iy1, 0.0)
    area_a = jnp.maximum(ax2 - ax1, 0.0) * jnp.maximum(ay2 - ay1, 0.0)
    area_b = jnp.maximum(bx2 - bx1, 0.0) * jnp.maximum(by2 - by1, 0.0)
    iou = inter / jnp.maximum(area_a + area_b - inter, 1e-6)

    p_count = jnp.sum(mf)
    npairs = jnp.maximum(p_count * (p_count - 1.0) * 0.5, 1.0)
    include = (p_count >= 2.0).astype(jnp.float32)
    scale = include / npairs

    row = jax.lax.broadcasted_iota(jnp.int32, (NPAD, NPAD), 0)
    col = jax.lax.broadcasted_iota(jnp.int32, (NPAD, NPAD), 1)
    tie = row < col

    def pair_sum(key, val):
        u = mf * jnp.exp(-GAMMA * val)
        v = mf * jnp.exp(GAMMA * val)
        kc = key.reshape(NPAD, 1)
        kr = key
        before = (kc > kr) | ((kc == kr) & tie)
        prod = u.reshape(NPAD, 1) * v
        return jnp.sum(jnp.where(before, prod, 0.0))

    s1 = pair_sum(iou, pos_prob)
    s2 = pair_sum(pos_prob, iou)

    f1_ref[...] += (s1 * scale).reshape(1, 1)
    f2_ref[...] += (s2 * scale).reshape(1, 1)
    cnt_ref[...] += include.reshape(1, 1)

    @pl.when(b == B - 1)
    def _final():
        c = cnt_ref[...]
        denom = jnp.maximum(c, 1.0)
        has = (c > 0.0).astype(jnp.float32)
        f1_ref[...] = f1_ref[...] / denom * has
        f2_ref[...] = f2_ref[...] / denom * has


def kernel(cls, label_cls, pred_loc, label_loc, shape):
    pad = NPAD - N
    cls1 = cls.reshape(B, N, 2)[:, :, 1]
    cls1 = jnp.pad(cls1, ((0, 0), (0, pad))).reshape(B, 1, NPAD)
    lab = label_cls.reshape(B, N)
    lab = jnp.pad(lab, ((0, 0), (0, pad))).reshape(B, 1, NPAD)
    ploc = jnp.pad(pred_loc.reshape(B, 4, N), ((0, 0), (0, 0), (0, pad)))
    lloc = jnp.pad(label_loc.reshape(B, 4, N), ((0, 0), (0, 0), (0, pad)))
    shp = jnp.pad(shape.reshape(4, N), ((0, 0), (0, pad)),
                  constant_values=1.0)

    f1, f2, _ = pl.pallas_call(
        _loss_kernel,
        grid=(B,),
        in_specs=[
            pl.BlockSpec((1, 1, NPAD), lambda b: (b, 0, 0)),
            pl.BlockSpec((1, 1, NPAD), lambda b: (b, 0, 0)),
            pl.BlockSpec((1, 4, NPAD), lambda b: (b, 0, 0)),
            pl.BlockSpec((1, 4, NPAD), lambda b: (b, 0, 0)),
            pl.BlockSpec((4, NPAD), lambda b: (0, 0)),
        ],
        out_specs=[
            pl.BlockSpec((1, 1), lambda b: (0, 0)),
            pl.BlockSpec((1, 1), lambda b: (0, 0)),
            pl.BlockSpec((1, 1), lambda b: (0, 0)),
        ],
        out_shape=[
            jax.ShapeDtypeStruct((1, 1), jnp.float32),
            jax.ShapeDtypeStruct((1, 1), jnp.float32),
            jax.ShapeDtypeStruct((1, 1), jnp.float32),
        ],
    )(cls1, lab, ploc, lloc, shp)
    return (f1.reshape(()), f2.reshape(()))
